# baseline (speedup 1.0000x reference)
import jax
import jax.numpy as jnp
from jax import lax
from jax.experimental import pallas as pl
from jax.experimental.pallas import tpu as pltpu
from jax.experimental.pallas import tpu_sc as plsc

_B = 16384
_D = 64
_NC = 2
_NS = 16
_NW = _NC * _NS
_BPW = _B // _NW
_CH = 128
_NCH = _BPW // _CH
_G = 16
_NG = _CH // _G
_BV = 16384


def _tp_body(x_ref, b_ref, o_ref):
    x = x_ref[...]
    b = b_ref[...]
    o_ref[:, 0:_D] = jnp.swapaxes(x, 0, 1)
    o_ref[:, _D:2 * _D] = jnp.broadcast_to(jnp.swapaxes(b, 0, 1), (_BV, _D))


def _tc_transpose(table_t, bias_t):
    v = table_t.shape[1]
    grid = (v + _BV - 1) // _BV
    return pl.pallas_call(
        _tp_body,
        grid=(grid,),
        in_specs=[pl.BlockSpec((_D, _BV), lambda i: (0, i)),
                  pl.BlockSpec((1, _BV), lambda i: (0, i))],
        out_specs=pl.BlockSpec((_BV, 2 * _D), lambda i: (i, 0)),
        out_shape=jax.ShapeDtypeStruct((v, 2 * _D), jnp.float32),
    )(table_t, bias_t)


def _sc_body(user_id, film_id, user_table, film_table,
             out, idx_u, idx_f, u_rows, f_rows, out_v, sem):
    wid = lax.axis_index("s") * _NC + lax.axis_index("c")
    base = wid * _BPW

    pltpu.sync_copy(user_id.at[pl.ds(base, _BPW)], idx_u)
    pltpu.sync_copy(film_id.at[pl.ds(base, _BPW)], idx_f)

    for c in range(_NCH):
        sl = pl.ds(c * _CH, _CH)
        cp = [
            pltpu.async_copy(user_table.at[idx_u.at[sl]], u_rows, sem),
            pltpu.async_copy(film_table.at[idx_f.at[sl]], f_rows, sem),
        ]
        for h in cp:
            h.wait()

        def group(g, carry):
            rows = g * _G + lax.iota(jnp.int32, _G)
            sl16 = pl.ds(c * _CH + g * _G, _G)
            rot = lax.iota(jnp.int32, _G)
            bcol = jnp.full((_G,), _D, jnp.int32)
            acc = plsc.load_gather(u_rows, [rows, bcol]) + \
                plsc.load_gather(f_rows, [rows, bcol])
            for d in range(_D):
                uu = plsc.load_gather(u_rows, [rows, rot])
                ff = plsc.load_gather(f_rows, [rows, rot])
                acc = acc + uu * ff
                rot = (rot + 1) & (_D - 1)
            out_v[sl16] = acc
            return carry

        lax.fori_loop(0, _NG, group, 0)

    pltpu.sync_copy(out_v, out.at[pl.ds(base, _BPW)])


@jax.jit
def _run(user_id, film_id, user_table_t, film_table_t, user_bias_t,
         film_bias_t):
    ut128 = _tc_transpose(user_table_t, user_bias_t)
    ft128 = _tc_transpose(film_table_t, film_bias_t)
    mesh = plsc.VectorSubcoreMesh(core_axis_name="c", subcore_axis_name="s")
    f = pl.kernel(
        _sc_body,
        out_type=jax.ShapeDtypeStruct((_B,), jnp.float32),
        mesh=mesh,
        compiler_params=pltpu.CompilerParams(needs_layout_passes=False),
        scratch_types=[
            pltpu.VMEM((_BPW,), jnp.int32),
            pltpu.VMEM((_BPW,), jnp.int32),
            pltpu.VMEM((_CH, 2 * _D), jnp.float32),
            pltpu.VMEM((_CH, 2 * _D), jnp.float32),
            pltpu.VMEM((_BPW,), jnp.float32),
            pltpu.SemaphoreType.DMA,
        ],
    )
    return f(user_id, film_id, ut128, ft128)


def kernel(user_id, film_id, user_table, film_table, user_bias_table,
           film_bias_table):
    return _run(user_id, film_id, user_table.T, film_table.T,
                user_bias_table.T, film_bias_table.T)


# speedup vs baseline: 1.6096x; 1.6096x over previous
import jax
import jax.numpy as jnp
from jax import lax
from jax.experimental import pallas as pl
from jax.experimental.pallas import tpu as pltpu
from jax.experimental.pallas import tpu_sc as plsc

_B = 16384
_D = 64
_NC = 2
_NS = 16
_NW = _NC * _NS
_BPW = _B // _NW
_CH = 128
_NCH = _BPW // _CH
_G = 16
_NG = _CH // _G
_BV = 16384


def _tp_body(x_ref, o_ref):
    x = x_ref[...]
    o_ref[:, 0:_D] = jnp.swapaxes(x, 0, 1)
    o_ref[:, _D:2 * _D] = jnp.zeros((_BV, _D), jnp.float32)


def _tc_transpose(table_t):
    v = table_t.shape[1]
    grid = (v + _BV - 1) // _BV
    return pl.pallas_call(
        _tp_body,
        grid=(grid,),
        in_specs=[pl.BlockSpec((_D, _BV), lambda i: (0, i))],
        out_specs=pl.BlockSpec((_BV, 2 * _D), lambda i: (i, 0)),
        out_shape=jax.ShapeDtypeStruct((v, 2 * _D), jnp.float32),
    )(table_t)


def _sc_body(user_id, film_id, user_table, film_table, user_bias, film_bias,
             out, idx_u, idx_f, u_rows, f_rows, ub_v, fb_v, out_v, sem):
    wid = lax.axis_index("s") * _NC + lax.axis_index("c")
    base = wid * _BPW

    pltpu.sync_copy(user_id.at[pl.ds(base, _BPW)], idx_u)
    pltpu.sync_copy(film_id.at[pl.ds(base, _BPW)], idx_f)

    for c in range(_NCH):
        sl = pl.ds(c * _CH, _CH)
        cp = [
            pltpu.async_copy(user_table.at[idx_u.at[sl]], u_rows, sem),
            pltpu.async_copy(film_table.at[idx_f.at[sl]], f_rows, sem),
            pltpu.async_copy(user_bias.at[0].at[idx_u.at[sl]], ub_v, sem),
            pltpu.async_copy(film_bias.at[0].at[idx_f.at[sl]], fb_v, sem),
        ]
        for h in cp:
            h.wait()

        def group(g, carry):
            rows = g * _G + lax.iota(jnp.int32, _G)
            sl16 = pl.ds(c * _CH + g * _G, _G)
            rot = lax.iota(jnp.int32, _G)
            slg = pl.ds(g * _G, _G)
            acc = ub_v[slg] + fb_v[slg]
            for d in range(_D):
                uu = plsc.load_gather(u_rows, [rows, rot])
                ff = plsc.load_gather(f_rows, [rows, rot])
                acc = acc + uu * ff
                rot = (rot + 1) & (_D - 1)
            out_v[sl16] = acc
            return carry

        lax.fori_loop(0, _NG, group, 0)

    pltpu.sync_copy(out_v, out.at[pl.ds(base, _BPW)])


@jax.jit
def _run(user_id, film_id, user_table_t, film_table_t, user_bias_t,
         film_bias_t):
    ut128 = _tc_transpose(user_table_t)
    ft128 = _tc_transpose(film_table_t)
    mesh = plsc.VectorSubcoreMesh(core_axis_name="c", subcore_axis_name="s")
    f = pl.kernel(
        _sc_body,
        out_type=jax.ShapeDtypeStruct((_B,), jnp.float32),
        mesh=mesh,
        compiler_params=pltpu.CompilerParams(needs_layout_passes=False),
        scratch_types=[
            pltpu.VMEM((_BPW,), jnp.int32),
            pltpu.VMEM((_BPW,), jnp.int32),
            pltpu.VMEM((_CH, 2 * _D), jnp.float32),
            pltpu.VMEM((_CH, 2 * _D), jnp.float32),
            pltpu.VMEM((_CH,), jnp.float32),
            pltpu.VMEM((_CH,), jnp.float32),
            pltpu.VMEM((_BPW,), jnp.float32),
            pltpu.SemaphoreType.DMA,
        ],
    )
    return f(user_id, film_id, ut128, ft128, user_bias_t, film_bias_t)


def kernel(user_id, film_id, user_table, film_table, user_bias_table,
           film_bias_table):
    return _run(user_id, film_id, user_table.T, film_table.T,
                user_bias_table.T, film_bias_table.T)


# speedup vs baseline: 1.6354x; 1.0160x over previous
import jax
import jax.numpy as jnp
from jax import lax
from jax.experimental import pallas as pl
from jax.experimental.pallas import tpu as pltpu
from jax.experimental.pallas import tpu_sc as plsc

_B = 16384
_D = 64
_NC = 2
_NS = 16
_NW = _NC * _NS
_BPW = _B // _NW
_CH = 128
_NCH = _BPW // _CH
_G = 16
_NG = _CH // _G
_BV = 24576


def _tp_body(x_ref, o_ref):
    x = x_ref[...]
    o_ref[:, 0:_D] = jnp.swapaxes(x, 0, 1)
    o_ref[:, _D:2 * _D] = jnp.zeros((_BV, _D), jnp.float32)


def _tc_transpose(table_t):
    v = table_t.shape[1]
    grid = (v + _BV - 1) // _BV
    return pl.pallas_call(
        _tp_body,
        grid=(grid,),
        in_specs=[pl.BlockSpec((_D, _BV), lambda i: (0, i))],
        out_specs=pl.BlockSpec((_BV, 2 * _D), lambda i: (i, 0)),
        out_shape=jax.ShapeDtypeStruct((v, 2 * _D), jnp.float32),
    )(table_t)


def _sc_body(user_id, film_id, user_table, film_table, user_bias, film_bias,
             out, idx_u, idx_f, u_rows, f_rows, ub_v, fb_v, out_v, sem):
    wid = lax.axis_index("s") * _NC + lax.axis_index("c")
    base = wid * _BPW

    pltpu.sync_copy(user_id.at[pl.ds(base, _BPW)], idx_u)
    pltpu.sync_copy(film_id.at[pl.ds(base, _BPW)], idx_f)

    for c in range(_NCH):
        sl = pl.ds(c * _CH, _CH)
        cp = [
            pltpu.async_copy(user_table.at[idx_u.at[sl]], u_rows, sem),
            pltpu.async_copy(film_table.at[idx_f.at[sl]], f_rows, sem),
            pltpu.async_copy(user_bias.at[0].at[idx_u.at[sl]], ub_v, sem),
            pltpu.async_copy(film_bias.at[0].at[idx_f.at[sl]], fb_v, sem),
        ]
        for h in cp:
            h.wait()

        def group(g, carry):
            rows = g * _G + lax.iota(jnp.int32, _G)
            sl16 = pl.ds(c * _CH + g * _G, _G)
            rot = lax.iota(jnp.int32, _G)
            slg = pl.ds(g * _G, _G)
            acc = ub_v[slg] + fb_v[slg]
            for d in range(_D):
                uu = plsc.load_gather(u_rows, [rows, rot])
                ff = plsc.load_gather(f_rows, [rows, rot])
                acc = acc + uu * ff
                rot = (rot + 1) & (_D - 1)
            out_v[sl16] = acc
            return carry

        lax.fori_loop(0, _NG, group, 0)

    pltpu.sync_copy(out_v, out.at[pl.ds(base, _BPW)])


@jax.jit
def _run(user_id, film_id, user_table_t, film_table_t, user_bias_t,
         film_bias_t):
    ut128 = _tc_transpose(user_table_t)
    ft128 = _tc_transpose(film_table_t)
    mesh = plsc.VectorSubcoreMesh(core_axis_name="c", subcore_axis_name="s")
    f = pl.kernel(
        _sc_body,
        out_type=jax.ShapeDtypeStruct((_B,), jnp.float32),
        mesh=mesh,
        compiler_params=pltpu.CompilerParams(needs_layout_passes=False),
        scratch_types=[
            pltpu.VMEM((_BPW,), jnp.int32),
            pltpu.VMEM((_BPW,), jnp.int32),
            pltpu.VMEM((_CH, 2 * _D), jnp.float32),
            pltpu.VMEM((_CH, 2 * _D), jnp.float32),
            pltpu.VMEM((_CH,), jnp.float32),
            pltpu.VMEM((_CH,), jnp.float32),
            pltpu.VMEM((_BPW,), jnp.float32),
            pltpu.SemaphoreType.DMA,
        ],
    )
    return f(user_id, film_id, ut128, ft128, user_bias_t, film_bias_t)


def kernel(user_id, film_id, user_table, film_table, user_bias_table,
           film_bias_table):
    return _run(user_id, film_id, user_table.T, film_table.T,
                user_bias_table.T, film_bias_table.T)


# speedup vs baseline: 1.6558x; 1.0125x over previous
import jax
import jax.numpy as jnp
from jax import lax
from jax.experimental import pallas as pl
from jax.experimental.pallas import tpu as pltpu
from jax.experimental.pallas import tpu_sc as plsc

_B = 16384
_D = 64
_NC = 2
_NS = 16
_NW = _NC * _NS
_BPW = _B // _NW
_CH = 128
_NCH = _BPW // _CH
_G = 16
_NG = _CH // _G
_BV = 24576


def _tp_body(x_ref, o_ref):
    x = x_ref[...]
    o_ref[:, 0:_D] = jnp.swapaxes(x, 0, 1)


def _tc_transpose(table_t):
    v = table_t.shape[1]
    grid = (v + _BV - 1) // _BV
    return pl.pallas_call(
        _tp_body,
        grid=(grid,),
        in_specs=[pl.BlockSpec((_D, _BV), lambda i: (0, i))],
        out_specs=pl.BlockSpec((_BV, 2 * _D), lambda i: (i, 0)),
        out_shape=jax.ShapeDtypeStruct((v, 2 * _D), jnp.float32),
    )(table_t)


def _sc_body(user_id, film_id, user_table, film_table, user_bias, film_bias,
             out, idx_u, idx_f, u_rows, f_rows, ub_v, fb_v,
             u_rows2, f_rows2, ub_v2, fb_v2, out_v, sem, sem2):
    wid = lax.axis_index("s") * _NC + lax.axis_index("c")
    base = wid * _BPW

    pltpu.sync_copy(user_id.at[pl.ds(base, _BPW)], idx_u)
    pltpu.sync_copy(film_id.at[pl.ds(base, _BPW)], idx_f)

    bufs = [(u_rows, f_rows, ub_v, fb_v, sem),
            (u_rows2, f_rows2, ub_v2, fb_v2, sem2)]

    def fire(c):
        sl = pl.ds(c * _CH, _CH)
        bu, bf, bub, bfb, bs = bufs[c % 2]
        return [
            pltpu.async_copy(user_table.at[idx_u.at[sl]], bu, bs),
            pltpu.async_copy(film_table.at[idx_f.at[sl]], bf, bs),
            pltpu.async_copy(user_bias.at[0].at[idx_u.at[sl]], bub, bs),
            pltpu.async_copy(film_bias.at[0].at[idx_f.at[sl]], bfb, bs),
        ]

    cp = fire(0)
    for c in range(_NCH):
        u_c, f_c, ub_c, fb_c, _ = bufs[c % 2]
        for h in cp:
            h.wait()
        if c + 1 < _NCH:
            cp = fire(c + 1)

        def group(g, carry):
            rows = g * _G + lax.iota(jnp.int32, _G)
            sl16 = pl.ds(c * _CH + g * _G, _G)
            rot = lax.iota(jnp.int32, _G)
            slg = pl.ds(g * _G, _G)
            acc = ub_c[slg] + fb_c[slg]
            for d in range(_D):
                uu = plsc.load_gather(u_c, [rows, rot])
                ff = plsc.load_gather(f_c, [rows, rot])
                acc = acc + uu * ff
                rot = (rot + 1) & (_D - 1)
            out_v[sl16] = acc
            return carry

        lax.fori_loop(0, _NG, group, 0)

    pltpu.sync_copy(out_v, out.at[pl.ds(base, _BPW)])


@jax.jit
def _run(user_id, film_id, user_table_t, film_table_t, user_bias_t,
         film_bias_t):
    ut128 = _tc_transpose(user_table_t)
    ft128 = _tc_transpose(film_table_t)
    mesh = plsc.VectorSubcoreMesh(core_axis_name="c", subcore_axis_name="s")
    f = pl.kernel(
        _sc_body,
        out_type=jax.ShapeDtypeStruct((_B,), jnp.float32),
        mesh=mesh,
        compiler_params=pltpu.CompilerParams(needs_layout_passes=False),
        scratch_types=[
            pltpu.VMEM((_BPW,), jnp.int32),
            pltpu.VMEM((_BPW,), jnp.int32),
            pltpu.VMEM((_CH, 2 * _D), jnp.float32),
            pltpu.VMEM((_CH, 2 * _D), jnp.float32),
            pltpu.VMEM((_CH,), jnp.float32),
            pltpu.VMEM((_CH,), jnp.float32),
            pltpu.VMEM((_CH, 2 * _D), jnp.float32),
            pltpu.VMEM((_CH, 2 * _D), jnp.float32),
            pltpu.VMEM((_CH,), jnp.float32),
            pltpu.VMEM((_CH,), jnp.float32),
            pltpu.VMEM((_BPW,), jnp.float32),
            pltpu.SemaphoreType.DMA,
            pltpu.SemaphoreType.DMA,
        ],
    )
    return f(user_id, film_id, ut128, ft128, user_bias_t, film_bias_t)


def kernel(user_id, film_id, user_table, film_table, user_bias_table,
           film_bias_table):
    return _run(user_id, film_id, user_table.T, film_table.T,
                user_bias_table.T, film_bias_table.T)
